# baseline (speedup 1.0000x reference)
import functools

import jax
import jax.numpy as jnp
from jax import lax
from jax.experimental import pallas as pl
from jax.experimental.pallas import tpu as pltpu
from jax.experimental.pallas import tpu_sc as plsc

D = 64
NW = 32
CHUNK = 256
L = 16


def _gather_kernel(n_chunks, bchunks, table_hbm, idx_hbm, out_hbm,
                   idx_v, gbuf, obuf, gsem0, gsem1, wsem0, wsem1):
    wid = lax.axis_index("s") * 2 + lax.axis_index("c")
    base_chunk = wid * n_chunks
    n_idx = n_chunks * CHUNK

    pltpu.sync_copy(idx_hbm.at[pl.ds(base_chunk * CHUNK, n_idx)], idx_v)

    gsems = (gsem0, gsem1)
    wsems = (wsem0, wsem1)
    iota = lax.iota(jnp.int32, L)

    def start_gather(jj, b):
        pltpu.async_copy(
            table_hbm.at[idx_v.at[pl.ds(jj * CHUNK, CHUNK)]], gbuf.at[b],
            gsems[b])

    def wait_write(b):
        pltpu.make_async_copy(
            obuf.at[b], out_hbm.at[pl.ds(0, 8), pl.ds(0, 2), :], wsems[b]
        ).wait()

    def process(jj, b):
        k = base_chunk + jj
        s = k // bchunks
        bh0 = (k % bchunks) * (CHUNK // 128)

        pltpu.make_async_copy(
            table_hbm.at[idx_v.at[pl.ds(jj * CHUNK, CHUNK)]],
            gbuf.at[b], gsems[b]).wait()

        def tr_body(t, carry):
            dh = t >> 1
            q = t & 1
            rbase = iota + q * 128
            for dl in range(8):
                col = jnp.full((L,), 8 * dh + dl, jnp.int32)
                for blg in range(128 // L):
                    x = plsc.load_gather(gbuf.at[b], [rbase + blg * L, col])
                    obuf[b, dh, q, pl.ds(dl * 128 + blg * L, L)] = x
            return carry

        lax.fori_loop(0, 16, tr_body, 0, unroll=False)

        pltpu.async_copy(
            obuf.at[b],
            out_hbm.at[pl.ds(s * 8, 8), pl.ds(bh0, 2), :], wsems[b])

    start_gather(0, 0)
    start_gather(1, 1)
    process(0, 0)
    start_gather(2, 0)
    process(1, 1)
    start_gather(3, 1)

    def body(c, _):
        for b in range(2):
            jj = 2 * c + b
            wait_write(b)
            process(jj, b)
            start_gather(jj + 2, b)
        return _

    lax.fori_loop(1, n_chunks // 2 - 1, body, 0, unroll=False)

    for b in range(2):
        jj = n_chunks - 2 + b
        wait_write(b)
        process(jj, b)
        wait_write(b)


def kernel(table, encoded_captions):
    B, S = encoded_captions.shape
    N = B * S
    assert B % (CHUNK * 2) == 0 and N % (NW * CHUNK * 2) == 0
    n_chunks = N // (NW * CHUNK)
    bchunks = B // CHUNK
    idx = encoded_captions.T.reshape(N).astype(jnp.int32)

    mesh = plsc.VectorSubcoreMesh(core_axis_name="c", subcore_axis_name="s")

    run = functools.partial(
        pl.kernel,
        out_type=jax.ShapeDtypeStruct((S * D // 8, B // 128, 1024),
                                      jnp.float32),
        mesh=mesh,
        compiler_params=pltpu.CompilerParams(use_tc_tiling_on_sc=False,
                                             needs_layout_passes=False),
        scratch_types=[
            pltpu.VMEM((N // NW,), jnp.int32),
            pltpu.VMEM((2, CHUNK, D), jnp.float32),
            pltpu.VMEM((2, D // 8, CHUNK // 128, 1024), jnp.float32),
            pltpu.SemaphoreType.DMA,
            pltpu.SemaphoreType.DMA,
            pltpu.SemaphoreType.DMA,
            pltpu.SemaphoreType.DMA,
        ],
    )(functools.partial(_gather_kernel, n_chunks, bchunks))

    out3 = run(table, idx)
    out5 = out3.reshape(S, D // 8, B // 128, 8, 128)
    return out5.transpose(0, 1, 3, 2, 4).reshape(S, D, B).transpose(2, 0, 1)


# speedup vs baseline: 1.4568x; 1.4568x over previous
import functools

import jax
import jax.numpy as jnp
from jax import lax
from jax.experimental import pallas as pl
from jax.experimental.pallas import tpu as pltpu
from jax.experimental.pallas import tpu_sc as plsc

D = 64
NW = 32
CHUNK = 512


def _gather_kernel(n_chunks, table_hbm, idx_hbm, out_hbm, idx_v, rows_v,
                   gsem0, gsem1):
    wid = lax.axis_index("s") * 2 + lax.axis_index("c")
    n_idx = n_chunks * CHUNK
    base_row = wid * n_idx

    pltpu.sync_copy(idx_hbm.at[pl.ds(base_row, n_idx)], idx_v)

    gsems = (gsem0, gsem1)

    for b in range(2):
        pltpu.async_copy(
            table_hbm.at[idx_v.at[pl.ds(b * CHUNK, CHUNK)]], rows_v.at[b],
            gsems[b])

    def body(c, _):
        for b in range(2):
            cc = 2 * c + b
            pltpu.make_async_copy(
                table_hbm.at[idx_v.at[pl.ds(cc * CHUNK, CHUNK)]],
                rows_v.at[b], gsems[b]).wait()
            pltpu.sync_copy(rows_v.at[b],
                            out_hbm.at[pl.ds(base_row + cc * CHUNK, CHUNK)])
            pltpu.async_copy(
                table_hbm.at[idx_v.at[pl.ds((cc + 2) * CHUNK, CHUNK)]],
                rows_v.at[b], gsems[b])
        return _

    lax.fori_loop(0, n_chunks // 2 - 1, body, 0, unroll=False)

    for b in range(2):
        cc = n_chunks - 2 + b
        pltpu.make_async_copy(
            table_hbm.at[idx_v.at[pl.ds(cc * CHUNK, CHUNK)]],
            rows_v.at[b], gsems[b]).wait()
        pltpu.sync_copy(rows_v.at[b],
                        out_hbm.at[pl.ds(base_row + cc * CHUNK, CHUNK)])


def kernel(table, encoded_captions):
    B, S = encoded_captions.shape
    N = B * S
    assert N % (NW * CHUNK * 2) == 0
    n_chunks = N // (NW * CHUNK)
    idx = encoded_captions.reshape(N).astype(jnp.int32)

    mesh = plsc.VectorSubcoreMesh(core_axis_name="c", subcore_axis_name="s")

    run = functools.partial(
        pl.kernel,
        out_type=jax.ShapeDtypeStruct((N, D), jnp.float32),
        mesh=mesh,
        compiler_params=pltpu.CompilerParams(use_tc_tiling_on_sc=False),
        scratch_types=[
            pltpu.VMEM((N // NW,), jnp.int32),
            pltpu.VMEM((2, CHUNK, D), jnp.float32),
            pltpu.SemaphoreType.DMA,
            pltpu.SemaphoreType.DMA,
        ],
    )(functools.partial(_gather_kernel, n_chunks))

    out = run(table, idx)
    return out.reshape(B, S, D)
